# Initial kernel scaffold; baseline (speedup 1.0000x reference)
import jax
import jax.numpy as jnp
from jax.experimental import pallas as pl


def kernel(x, edge_index, W1, b1, W2, b2, W3, b3, Wres, bres):
    raise NotImplementedError("write your pallas kernel here")


# speedup vs baseline: 10.9669x; 10.9669x over previous
import functools

import jax
import jax.numpy as jnp
from jax import lax
from jax.experimental import pallas as pl
from jax.experimental.pallas import tpu as pltpu
from jax.experimental.pallas import tpu_sc as plsc

N = 10000
E = 320000
D = 128
NPAD = 10240
NC = 2
NS = 16
NW = NC * NS
EPT = E // NW
CHUNK = 80
NCHUNK = EPT // CHUNK
RPT = NPAD // NS

_mesh = plsc.VectorSubcoreMesh(core_axis_name="c", subcore_axis_name="s")


@functools.partial(
    pl.kernel, mesh=_mesh,
    out_type=jax.ShapeDtypeStruct((NC, NPAD, 16), jnp.float32),
    scratch_types=[
        pltpu.VMEM((CHUNK,), jnp.int32),
        pltpu.VMEM((CHUNK, 16), jnp.float32),
        pltpu.VMEM_SHARED((NPAD, 16), jnp.float32),
    ],
)
def _sc_degree(dst_hbm, zeros_hbm, ones_hbm, out_hbm, dst_v, ones_v, acc_sp):
    cid = lax.axis_index("c")
    sid = lax.axis_index("s")
    base = (cid * NS + sid) * EPT
    pltpu.sync_copy(ones_hbm, ones_v)
    pltpu.sync_copy(zeros_hbm, acc_sp.at[pl.ds(sid * RPT, RPT)])
    plsc.subcore_barrier()

    def body(c, carry):
        off = base + c * CHUNK
        pltpu.sync_copy(dst_hbm.at[pl.ds(off, CHUNK)], dst_v)
        pltpu.sync_copy(ones_v, acc_sp.at[dst_v], add=True)
        return carry

    lax.fori_loop(0, NCHUNK, body, 0)
    plsc.subcore_barrier()
    pltpu.sync_copy(acc_sp.at[pl.ds(sid * RPT, RPT)],
                    out_hbm.at[cid, pl.ds(sid * RPT, RPT)])


@functools.partial(
    pl.kernel, mesh=_mesh,
    out_type=jax.ShapeDtypeStruct((NC, NPAD, D), jnp.float32),
    scratch_types=[
        pltpu.VMEM((CHUNK,), jnp.int32),
        pltpu.VMEM((CHUNK,), jnp.int32),
        pltpu.VMEM((CHUNK, D), jnp.float32),
        pltpu.VMEM_SHARED((NPAD, D), jnp.float32),
    ],
)
def _sc_spmm(y_hbm, src_hbm, dst_hbm, zrows_hbm, out_hbm,
             src_v, dst_v, rows_v, acc_sp):
    cid = lax.axis_index("c")
    sid = lax.axis_index("s")
    base = (cid * NS + sid) * EPT

    @pl.when(cid == 0)
    def _():
        pltpu.sync_copy(y_hbm.at[pl.ds(sid * RPT, RPT)],
                        acc_sp.at[pl.ds(sid * RPT, RPT)])

    @pl.when(cid != 0)
    def _():
        pltpu.sync_copy(zrows_hbm, acc_sp.at[pl.ds(sid * RPT, RPT)])

    plsc.subcore_barrier()

    def body(c, carry):
        off = base + c * CHUNK
        pltpu.sync_copy(src_hbm.at[pl.ds(off, CHUNK)], src_v)
        pltpu.sync_copy(dst_hbm.at[pl.ds(off, CHUNK)], dst_v)
        pltpu.sync_copy(y_hbm.at[src_v], rows_v)
        pltpu.sync_copy(rows_v, acc_sp.at[dst_v], add=True)
        return carry

    lax.fori_loop(0, NCHUNK, body, 0)
    plsc.subcore_barrier()
    pltpu.sync_copy(acc_sp.at[pl.ds(sid * RPT, RPT)],
                    out_hbm.at[cid, pl.ds(sid * RPT, RPT)])


_R = 2048


def _row_spec():
    return pl.BlockSpec((_R, D), lambda i: (i, 0))


def _w_spec():
    return pl.BlockSpec((D, D), lambda i: (0, 0))


def _b_spec():
    return pl.BlockSpec((1, D), lambda i: (0, 0))


def _zp_spec():
    return pl.BlockSpec((NC, _R, D), lambda i: (0, i, 0))


def _tc_prep(xp, W1, Wres, bres2, degp):
    def body(x_ref, w1_ref, wr_ref, br_ref, dp_ref, y1_ref, res_ref, dinv_ref):
        deg = 1.0 + dp_ref[0, :, 0:1] + dp_ref[1, :, 0:1]
        dinv = lax.rsqrt(deg)
        xb = x_ref[...]
        y1_ref[...] = dinv * jnp.dot(xb, w1_ref[...],
                                     preferred_element_type=jnp.float32)
        res_ref[...] = jnp.dot(xb, wr_ref[...],
                               preferred_element_type=jnp.float32) + br_ref[...]
        dinv_ref[...] = jnp.broadcast_to(dinv, (_R, D))

    out = pl.pallas_call(
        body,
        grid=(NPAD // _R,),
        in_specs=[_row_spec(), _w_spec(), _w_spec(), _b_spec(),
                  pl.BlockSpec((NC, _R, 16), lambda i: (0, i, 0))],
        out_specs=[_row_spec(), _row_spec(), _row_spec()],
        out_shape=[jax.ShapeDtypeStruct((NPAD, D), jnp.float32)] * 3,
    )(xp, W1, Wres, bres2, degp)
    return out


def _tc_mid(zp, dinv, b2, res, Wn):
    with_res = res is not None

    def body(zp_ref, dinv_ref, b_ref, *rest):
        if with_res:
            res_ref, w_ref, out_ref = rest
        else:
            w_ref, out_ref = rest
        z = zp_ref[0] + zp_ref[1]
        pre = dinv_ref[...] * z + b_ref[...]
        if with_res:
            pre = pre + res_ref[...]
        h = jnp.maximum(pre, 0.0)
        out_ref[...] = dinv_ref[...] * jnp.dot(
            h, w_ref[...], preferred_element_type=jnp.float32)

    in_specs = [_zp_spec(), _row_spec(), _b_spec()]
    args = [zp, dinv, b2]
    if with_res:
        in_specs.append(_row_spec())
        args.append(res)
    in_specs.append(_w_spec())
    args.append(Wn)
    return pl.pallas_call(
        body,
        grid=(NPAD // _R,),
        in_specs=in_specs,
        out_specs=_row_spec(),
        out_shape=jax.ShapeDtypeStruct((NPAD, D), jnp.float32),
    )(*args)


def _tc_final(zp, dinv, b2):
    def body(zp_ref, dinv_ref, b_ref, out_ref):
        o = dinv_ref[...] * (zp_ref[0] + zp_ref[1]) + b_ref[...]
        m = jnp.max(o, axis=1, keepdims=True)
        s = o - m
        out_ref[...] = s - jnp.log(jnp.sum(jnp.exp(s), axis=1, keepdims=True))

    return pl.pallas_call(
        body,
        grid=(NPAD // _R,),
        in_specs=[_zp_spec(), _row_spec(), _b_spec()],
        out_specs=_row_spec(),
        out_shape=jax.ShapeDtypeStruct((NPAD, D), jnp.float32),
    )(zp, dinv, b2)


def kernel(x, edge_index, W1, b1, W2, b2, W3, b3, Wres, bres):
    xp = jnp.pad(x, ((0, NPAD - N), (0, 0)))
    src = edge_index[0]
    dst = edge_index[1]
    zeros16 = jnp.zeros((RPT, 16), jnp.float32)
    ones16 = jnp.ones((CHUNK, 16), jnp.float32)
    zrows = jnp.zeros((RPT, D), jnp.float32)

    degp = _sc_degree(dst, zeros16, ones16)
    y1, res, dinv = _tc_prep(xp, W1, Wres, bres.reshape(1, D), degp)
    z1 = _sc_spmm(y1, src, dst, zrows)
    y2 = _tc_mid(z1, dinv, b1.reshape(1, D), res, W2)
    z2 = _sc_spmm(y2, src, dst, zrows)
    y3 = _tc_mid(z2, dinv, b2.reshape(1, D), None, W3)
    z3 = _sc_spmm(y3, src, dst, zrows)
    out = _tc_final(z3, dinv, b3.reshape(1, D))
    return out[:N]


# speedup vs baseline: 20.5536x; 1.8741x over previous
import functools

import jax
import jax.numpy as jnp
from jax import lax
from jax.experimental import pallas as pl
from jax.experimental.pallas import tpu as pltpu
from jax.experimental.pallas import tpu_sc as plsc

N = 10000
E = 320000
D = 128
NPAD = 10240
NC = 2
NS = 16
NW = NC * NS
EPT = E // NW
CHUNK = 80
NCHUNK = EPT // CHUNK
SCHUNK = 40
SNCHUNK = EPT // SCHUNK
NBUF = 5
NDB = 2 * NBUF
NGRP = SNCHUNK // NBUF
RPT = NPAD // NS

_mesh = plsc.VectorSubcoreMesh(core_axis_name="c", subcore_axis_name="s")


@functools.partial(
    pl.kernel, mesh=_mesh,
    out_type=jax.ShapeDtypeStruct((NC, NPAD, 16), jnp.float32),
    scratch_types=[
        pltpu.VMEM((CHUNK,), jnp.int32),
        pltpu.VMEM((CHUNK, 16), jnp.float32),
        pltpu.VMEM_SHARED((NPAD, 16), jnp.float32),
    ],
)
def _sc_degree(dst_hbm, zeros_hbm, ones_hbm, out_hbm, dst_v, ones_v, acc_sp):
    cid = lax.axis_index("c")
    sid = lax.axis_index("s")
    base = (cid * NS + sid) * EPT
    pltpu.sync_copy(ones_hbm, ones_v)
    pltpu.sync_copy(zeros_hbm, acc_sp.at[pl.ds(sid * RPT, RPT)])
    plsc.subcore_barrier()

    def body(c, carry):
        off = base + c * CHUNK
        pltpu.sync_copy(dst_hbm.at[pl.ds(off, CHUNK)], dst_v)
        pltpu.sync_copy(ones_v, acc_sp.at[dst_v], add=True)
        return carry

    lax.fori_loop(0, NCHUNK, body, 0)
    plsc.subcore_barrier()
    pltpu.sync_copy(acc_sp.at[pl.ds(sid * RPT, RPT)],
                    out_hbm.at[cid, pl.ds(sid * RPT, RPT)])


@functools.partial(
    pl.kernel, mesh=_mesh,
    out_type=jax.ShapeDtypeStruct((NC, NPAD, D), jnp.float32),
    scratch_types=[
        pltpu.VMEM((NBUF, SCHUNK), jnp.int32),
        pltpu.VMEM((NDB, SCHUNK), jnp.int32),
        pltpu.VMEM((NBUF, SCHUNK, D), jnp.float32),
        pltpu.VMEM_SHARED((NPAD, D), jnp.float32),
        pltpu.SemaphoreType.DMA((NBUF,)),
        pltpu.SemaphoreType.DMA((NDB,)),
        pltpu.SemaphoreType.DMA((NBUF,)),
        pltpu.SemaphoreType.DMA((NBUF,)),
    ],
)
def _sc_spmm(y_hbm, src_hbm, dst_hbm, zrows_hbm, out_hbm,
             src_v, dst_v, rows_v, acc_sp, sem_is, sem_id, sem_g, sem_s):
    cid = lax.axis_index("c")
    sid = lax.axis_index("s")
    base = (cid * NS + sid) * EPT

    def start_src(c, b):
        pltpu.make_async_copy(src_hbm.at[pl.ds(base + c * SCHUNK, SCHUNK)],
                              src_v.at[b], sem_is.at[b]).start()

    def wait_src(b):
        pltpu.make_async_copy(src_hbm.at[pl.ds(0, SCHUNK)], src_v.at[b],
                              sem_is.at[b]).wait()

    def start_dst(c, db):
        pltpu.make_async_copy(dst_hbm.at[pl.ds(base + c * SCHUNK, SCHUNK)],
                              dst_v.at[db], sem_id.at[db]).start()

    def wait_dst(db):
        pltpu.make_async_copy(dst_hbm.at[pl.ds(0, SCHUNK)], dst_v.at[db],
                              sem_id.at[db]).wait()

    def start_gather(b):
        pltpu.make_async_copy(y_hbm.at[src_v.at[b]], rows_v.at[b],
                              sem_g.at[b]).start()

    def wait_gather(b):
        pltpu.make_async_copy(y_hbm.at[src_v.at[b]], rows_v.at[b],
                              sem_g.at[b]).wait()

    def start_scatter(b, db):
        pltpu.make_async_copy(rows_v.at[b], acc_sp.at[dst_v.at[db]],
                              sem_s.at[b]).start(add=True)

    def wait_scatter(b, db):
        pltpu.make_async_copy(rows_v.at[b], acc_sp.at[dst_v.at[db]],
                              sem_s.at[b]).wait()

    @pl.when(cid == 0)
    def _():
        pltpu.sync_copy(y_hbm.at[pl.ds(sid * RPT, RPT)],
                        acc_sp.at[pl.ds(sid * RPT, RPT)])

    @pl.when(cid != 0)
    def _():
        pltpu.sync_copy(zrows_hbm, acc_sp.at[pl.ds(sid * RPT, RPT)])

    plsc.subcore_barrier()

    for b in range(NBUF):
        start_src(b, b)
    for db in range(NDB):
        start_dst(db, db)
    wait_src(0)
    start_gather(0)

    def group(g, carry):
        for b in range(NBUF):
            nb = (b + 1) % NBUF
            c = g * NBUF + b
            db = lax.rem(c, NDB)
            fdb = lax.rem(c + NBUF + 1, NDB)

            if b < NBUF - 1:
                wait_src(nb)

                @pl.when(g > 0)
                def _():
                    wait_scatter(nb, fdb)

                    @pl.when(c + NBUF + 1 < SNCHUNK)
                    def _():
                        start_dst(c + NBUF + 1, fdb)

                start_gather(nb)
            else:
                @pl.when(g < NGRP - 1)
                def _():
                    wait_src(nb)
                    wait_scatter(nb, fdb)

                    @pl.when(c + NBUF + 1 < SNCHUNK)
                    def _():
                        start_dst(c + NBUF + 1, fdb)

                    start_gather(nb)

            wait_gather(b)

            @pl.when(g < NGRP - 1)
            def _():
                start_src((g + 1) * NBUF + b, b)

            wait_dst(db)
            start_scatter(b, db)
        return carry

    lax.fori_loop(0, NGRP, group, 0)

    for b in range(NBUF):
        wait_scatter(b, NBUF + b)

    plsc.subcore_barrier()
    pltpu.sync_copy(acc_sp.at[pl.ds(sid * RPT, RPT)],
                    out_hbm.at[cid, pl.ds(sid * RPT, RPT)])


_R = 2048


def _row_spec():
    return pl.BlockSpec((_R, D), lambda i: (i, 0))


def _w_spec():
    return pl.BlockSpec((D, D), lambda i: (0, 0))


def _b_spec():
    return pl.BlockSpec((1, D), lambda i: (0, 0))


def _zp_spec():
    return pl.BlockSpec((NC, _R, D), lambda i: (0, i, 0))


def _tc_prep(xp, W1, Wres, bres2, degp):
    def body(x_ref, w1_ref, wr_ref, br_ref, dp_ref, y1_ref, res_ref, dinv_ref):
        deg = 1.0 + dp_ref[0, :, 0:1] + dp_ref[1, :, 0:1]
        dinv = lax.rsqrt(deg)
        xb = x_ref[...]
        y1_ref[...] = dinv * jnp.dot(xb, w1_ref[...],
                                     preferred_element_type=jnp.float32)
        res_ref[...] = jnp.dot(xb, wr_ref[...],
                               preferred_element_type=jnp.float32) + br_ref[...]
        dinv_ref[...] = jnp.broadcast_to(dinv, (_R, D))

    out = pl.pallas_call(
        body,
        grid=(NPAD // _R,),
        in_specs=[_row_spec(), _w_spec(), _w_spec(), _b_spec(),
                  pl.BlockSpec((NC, _R, 16), lambda i: (0, i, 0))],
        out_specs=[_row_spec(), _row_spec(), _row_spec()],
        out_shape=[jax.ShapeDtypeStruct((NPAD, D), jnp.float32)] * 3,
    )(xp, W1, Wres, bres2, degp)
    return out


def _tc_mid(zp, dinv, b, res, Wn):
    with_res = res is not None

    def body(zp_ref, dinv_ref, b_ref, *rest):
        if with_res:
            res_ref, w_ref, out_ref = rest
        else:
            w_ref, out_ref = rest
        z = zp_ref[0] + zp_ref[1]
        pre = dinv_ref[...] * z + b_ref[...]
        if with_res:
            pre = pre + res_ref[...]
        h = jnp.maximum(pre, 0.0)
        out_ref[...] = dinv_ref[...] * jnp.dot(
            h, w_ref[...], preferred_element_type=jnp.float32)

    in_specs = [_zp_spec(), _row_spec(), _b_spec()]
    args = [zp, dinv, b]
    if with_res:
        in_specs.append(_row_spec())
        args.append(res)
    in_specs.append(_w_spec())
    args.append(Wn)
    return pl.pallas_call(
        body,
        grid=(NPAD // _R,),
        in_specs=in_specs,
        out_specs=_row_spec(),
        out_shape=jax.ShapeDtypeStruct((NPAD, D), jnp.float32),
    )(*args)


def _tc_final(zp, dinv, b):
    def body(zp_ref, dinv_ref, b_ref, out_ref):
        o = dinv_ref[...] * (zp_ref[0] + zp_ref[1]) + b_ref[...]
        m = jnp.max(o, axis=1, keepdims=True)
        s = o - m
        out_ref[...] = s - jnp.log(jnp.sum(jnp.exp(s), axis=1, keepdims=True))

    return pl.pallas_call(
        body,
        grid=(NPAD // _R,),
        in_specs=[_zp_spec(), _row_spec(), _b_spec()],
        out_specs=_row_spec(),
        out_shape=jax.ShapeDtypeStruct((NPAD, D), jnp.float32),
    )(zp, dinv, b)


def kernel(x, edge_index, W1, b1, W2, b2, W3, b3, Wres, bres):
    xp = jnp.pad(x, ((0, NPAD - N), (0, 0)))
    src = edge_index[0]
    dst = edge_index[1]
    zeros16 = jnp.zeros((RPT, 16), jnp.float32)
    ones16 = jnp.ones((CHUNK, 16), jnp.float32)
    zrows = jnp.zeros((RPT, D), jnp.float32)

    degp = _sc_degree(dst, zeros16, ones16)
    y1, res, dinv = _tc_prep(xp, W1, Wres, bres.reshape(1, D), degp)
    z1 = _sc_spmm(y1, src, dst, zrows)
    y2 = _tc_mid(z1, dinv, b1.reshape(1, D), res, W2)
    z2 = _sc_spmm(y2, src, dst, zrows)
    y3 = _tc_mid(z2, dinv, b2.reshape(1, D), None, W3)
    z3 = _sc_spmm(y3, src, dst, zrows)
    out = _tc_final(z3, dinv, b3.reshape(1, D))
    return out[:N]
